# Initial kernel scaffold; baseline (speedup 1.0000x reference)
import jax
import jax.numpy as jnp
from jax.experimental import pallas as pl


def kernel(indices, table):
    raise NotImplementedError("write your pallas kernel here")


# speedup vs baseline: 2.9882x; 2.9882x over previous
import functools

import jax
import jax.numpy as jnp
from jax import lax
from jax.experimental import pallas as pl
from jax.experimental.pallas import tpu as pltpu
from jax.experimental.pallas import tpu_sc as plsc

VOCAB = 100000
EMBED = 128
BATCH = 4096
HIST = 50

NC = 2
NS = 16
NW = NC * NS

B_FLAT = BATCH * HIST
B_PER_W = B_FLAT // NW
CHUNK = 128
N_CHUNKS = B_PER_W // CHUNK
IDX_ROWS_PER_W = B_PER_W // CHUNK


def _gather_body(table_hbm, idx_hbm, out_hbm, idx_v, buf_v, sem):
    wid = lax.axis_index("s") * NC + lax.axis_index("c")
    row_base = wid * B_PER_W

    pltpu.sync_copy(idx_hbm.at[wid], idx_v)

    def step(j, carry):
        pltpu.async_copy(table_hbm.at[idx_v.at[j]], buf_v, sem).wait()
        pltpu.sync_copy(buf_v, out_hbm.at[pl.ds(row_base + j * CHUNK, CHUNK)])
        return carry

    lax.fori_loop(0, N_CHUNKS, step, 0)


@functools.partial(jax.jit, static_argnames=())
def _embed_flat(idx2d, table):
    mesh = plsc.VectorSubcoreMesh(
        core_axis_name="c", subcore_axis_name="s", num_cores=NC, num_subcores=NS
    )
    return pl.kernel(
        _gather_body,
        out_type=jax.ShapeDtypeStruct((B_FLAT, EMBED), jnp.float32),
        mesh=mesh,
        scratch_types=[
            pltpu.VMEM((N_CHUNKS, CHUNK), jnp.int32),
            pltpu.VMEM((CHUNK, EMBED), jnp.float32),
            pltpu.SemaphoreType.DMA,
        ],
    )(table, idx2d)


def kernel(indices, table):
    idx3d = indices.reshape(NW, N_CHUNKS, CHUNK)
    out = _embed_flat(idx3d, table)
    return out.reshape(BATCH, HIST, EMBED)


# speedup vs baseline: 3.2946x; 1.1025x over previous
import functools

import jax
import jax.numpy as jnp
from jax import lax
from jax.experimental import pallas as pl
from jax.experimental.pallas import tpu as pltpu
from jax.experimental.pallas import tpu_sc as plsc

VOCAB = 100000
EMBED = 128
BATCH = 4096
HIST = 50

NC = 2
NS = 16
NW = NC * NS

B_FLAT = BATCH * HIST
B_PER_W = B_FLAT // NW
CHUNK = 64
N_CHUNKS = B_PER_W // CHUNK
K = 5
NGROUPS = N_CHUNKS // K


def _gather_body(table_hbm, idx_hbm, out_hbm, idx_v, buf_a, buf_b,
                 gsem_a, gsem_b, wsem_a, wsem_b):
    wid = lax.axis_index("s") * NC + lax.axis_index("c")
    row_base = wid * B_PER_W

    pltpu.sync_copy(idx_hbm.at[wid], idx_v)

    def fire_gathers(buf, sem, g):
        for b in range(K):
            pltpu.async_copy(table_hbm.at[idx_v.at[g * K + b]], buf.at[b], sem)

    def wait_gathers(buf, sem, g):
        for b in range(K):
            pltpu.make_async_copy(
                table_hbm.at[idx_v.at[g * K + b]], buf.at[b], sem).wait()

    def fire_writes(buf, sem, g):
        for b in range(K):
            pltpu.async_copy(
                buf.at[b],
                out_hbm.at[pl.ds(row_base + (g * K + b) * CHUNK, CHUNK)], sem)

    def wait_writes(buf, sem, g):
        for b in range(K):
            pltpu.make_async_copy(
                buf.at[b],
                out_hbm.at[pl.ds(row_base + (g * K + b) * CHUNK, CHUNK)],
                sem).wait()

    fire_gathers(buf_a, gsem_a, 0)
    wait_gathers(buf_a, gsem_a, 0)
    fire_writes(buf_a, wsem_a, 0)
    fire_gathers(buf_b, gsem_b, 1)

    def step(m, carry):
        g1 = 2 * m + 1
        wait_gathers(buf_b, gsem_b, g1)
        fire_writes(buf_b, wsem_b, g1)
        wait_writes(buf_a, wsem_a, g1 - 1)
        fire_gathers(buf_a, gsem_a, g1 + 1)

        g2 = 2 * m + 2
        wait_gathers(buf_a, gsem_a, g2)
        fire_writes(buf_a, wsem_a, g2)
        wait_writes(buf_b, wsem_b, g2 - 1)
        fire_gathers(buf_b, gsem_b, g2 + 1)
        return carry

    lax.fori_loop(0, NGROUPS // 2 - 1, step, 0)

    g_last = NGROUPS - 1
    wait_gathers(buf_b, gsem_b, g_last)
    fire_writes(buf_b, wsem_b, g_last)
    wait_writes(buf_a, wsem_a, g_last - 1)
    wait_writes(buf_b, wsem_b, g_last)


@jax.jit
def _embed_flat(idx3d, table):
    mesh = plsc.VectorSubcoreMesh(
        core_axis_name="c", subcore_axis_name="s", num_cores=NC, num_subcores=NS
    )
    return pl.kernel(
        _gather_body,
        out_type=jax.ShapeDtypeStruct((B_FLAT, EMBED), jnp.float32),
        mesh=mesh,
        scratch_types=[
            pltpu.VMEM((N_CHUNKS, CHUNK), jnp.int32),
            pltpu.VMEM((K, CHUNK, EMBED), jnp.float32),
            pltpu.VMEM((K, CHUNK, EMBED), jnp.float32),
            pltpu.SemaphoreType.DMA,
            pltpu.SemaphoreType.DMA,
            pltpu.SemaphoreType.DMA,
            pltpu.SemaphoreType.DMA,
        ],
    )(table, idx3d)


def kernel(indices, table):
    idx3d = indices.astype(jnp.int32).reshape(NW, N_CHUNKS, CHUNK)
    out = _embed_flat(idx3d, table)
    return out.reshape(BATCH, HIST, EMBED)


# speedup vs baseline: 5.8892x; 1.7875x over previous
import jax
import jax.numpy as jnp
from jax import lax
from jax.experimental import pallas as pl
from jax.experimental.pallas import tpu as pltpu
from jax.experimental.pallas import tpu_sc as plsc

VOCAB = 100000
EMBED = 128
BATCH = 4096
HIST = 50

NC = 2
NS = 16
NW = NC * NS

B_PER_W = BATCH // NW
G = 8
NGROUPS = B_PER_W // G


def _gather_body(table_hbm, idx_hbm, out_hbm, idx_v, buf_a, buf_b,
                 gsem_a, gsem_b, wsem_a, wsem_b):
    wid = lax.axis_index("s") * NC + lax.axis_index("c")
    batch_base = wid * B_PER_W

    pltpu.sync_copy(idx_hbm.at[pl.ds(batch_base, B_PER_W)], idx_v)

    def fire_gathers(buf, sem, g):
        for b in range(G):
            pltpu.async_copy(table_hbm.at[idx_v.at[g * G + b]], buf.at[b], sem)

    def wait_gathers(buf, sem, g):
        for b in range(G):
            pltpu.make_async_copy(
                table_hbm.at[idx_v.at[g * G + b]], buf.at[b], sem).wait()

    def fire_write(buf, sem, g):
        pltpu.async_copy(buf, out_hbm.at[pl.ds(batch_base + g * G, G)], sem)

    def wait_write(buf, sem, g):
        pltpu.make_async_copy(
            buf, out_hbm.at[pl.ds(batch_base + g * G, G)], sem).wait()

    fire_gathers(buf_a, gsem_a, 0)
    wait_gathers(buf_a, gsem_a, 0)
    fire_write(buf_a, wsem_a, 0)
    fire_gathers(buf_b, gsem_b, 1)

    def step(m, carry):
        g1 = 2 * m + 1
        wait_gathers(buf_b, gsem_b, g1)
        fire_write(buf_b, wsem_b, g1)
        wait_write(buf_a, wsem_a, g1 - 1)
        fire_gathers(buf_a, gsem_a, g1 + 1)

        g2 = 2 * m + 2
        wait_gathers(buf_a, gsem_a, g2)
        fire_write(buf_a, wsem_a, g2)
        wait_write(buf_b, wsem_b, g2 - 1)
        fire_gathers(buf_b, gsem_b, g2 + 1)
        return carry

    lax.fori_loop(0, NGROUPS // 2 - 1, step, 0)

    g_last = NGROUPS - 1
    wait_gathers(buf_b, gsem_b, g_last)
    fire_write(buf_b, wsem_b, g_last)
    wait_write(buf_a, wsem_a, g_last - 1)
    wait_write(buf_b, wsem_b, g_last)


@jax.jit
def _embed(indices, table):
    mesh = plsc.VectorSubcoreMesh(
        core_axis_name="c", subcore_axis_name="s", num_cores=NC, num_subcores=NS
    )
    return pl.kernel(
        _gather_body,
        out_type=jax.ShapeDtypeStruct((BATCH, HIST, EMBED), jnp.float32),
        mesh=mesh,
        scratch_types=[
            pltpu.VMEM((B_PER_W, HIST), jnp.int32),
            pltpu.VMEM((G, HIST, EMBED), jnp.float32),
            pltpu.VMEM((G, HIST, EMBED), jnp.float32),
            pltpu.SemaphoreType.DMA,
            pltpu.SemaphoreType.DMA,
            pltpu.SemaphoreType.DMA,
            pltpu.SemaphoreType.DMA,
        ],
    )(table, indices)


def kernel(indices, table):
    return _embed(indices, table)
